# baseline (speedup 1.0000x reference)
import functools

import jax
import jax.numpy as jnp
from jax import lax
from jax.experimental import pallas as pl
from jax.experimental.pallas import tpu as pltpu
from jax.experimental.pallas import tpu_sc as plsc

N = 100000
E = 3200000
F = 16

CHUNK = 128
K = 4
GSZ = K * CHUNK
NCORE, NSUB = 2, 16
NW = NCORE * NSUB
EG = E // GSZ
NBIG = 21
GBIG, GSMALL = 196, 194
NPAD = 100096
RPT = NPAD // NSUB
ZROWS = 391

P = N * F // 128
PALL = NPAD * F // 128
BP = 3128


def _edge_pass(h, row2d, col2d, with_gather):
  mesh = plsc.VectorSubcoreMesh(core_axis_name="c", subcore_axis_name="s")

  @functools.partial(
      pl.kernel,
      out_type=jax.ShapeDtypeStruct((NCORE, NPAD, F), jnp.float32),
      mesh=mesh,
      scratch_types=[
          pltpu.VMEM((2, GSZ), jnp.int32),
          pltpu.VMEM((2, GSZ), jnp.int32),
          pltpu.VMEM((2, GSZ, F), jnp.float32),
          pltpu.VMEM((ZROWS, F), jnp.float32),
          pltpu.VMEM_SHARED((NPAD, F), jnp.float32),
          pltpu.SemaphoreType.DMA,
          pltpu.SemaphoreType.DMA,
          pltpu.SemaphoreType.DMA,
          pltpu.SemaphoreType.DMA,
          pltpu.SemaphoreType.DMA,
          pltpu.SemaphoreType.DMA,
      ],
      compiler_params=pltpu.CompilerParams(use_tc_tiling_on_sc=False),
  )
  def k(h_hbm, row_hbm, col_hbm, out_hbm, ridx, cidx, msg, zbuf, agg,
        gsem0, gsem1, ssem0, ssem1, isem0, isem1):
    c = lax.axis_index("c")
    s = lax.axis_index("s")
    wid = s * NCORE + c

    gw = GBIG - 2 * (wid >= NBIG).astype(jnp.int32)
    base = GSMALL * wid + 2 * jnp.minimum(wid, NBIG)
    gmax = base + gw - 1
    gsems = (gsem0, gsem1)
    ssems = (ssem0, ssem1)
    isems = (isem0, isem1)

    def pidx(b, g):
      off = jnp.minimum(base + g, gmax)
      if with_gather:
        pltpu.make_async_copy(row_hbm.at[off], ridx.at[b], isems[b]).start()
      pltpu.make_async_copy(col_hbm.at[off], cidx.at[b], isems[b]).start()

    def widx(b):
      if with_gather:
        pltpu.make_async_copy(row_hbm.at[0], ridx.at[b], isems[b]).wait()
      pltpu.make_async_copy(col_hbm.at[0], cidx.at[b], isems[b]).wait()

    def gat(b):
      if with_gather:
        pltpu.make_async_copy(h_hbm.at[ridx.at[b]], msg.at[b],
                              gsems[b]).start()

    def wgat(b):
      if with_gather:
        pltpu.make_async_copy(h_hbm.at[ridx.at[b]], msg.at[b],
                              gsems[b]).wait()

    def sca(b):
      pltpu.make_async_copy(msg.at[b], agg.at[cidx.at[b]],
                            ssems[b]).start(add=True)

    def wsca(b):
      pltpu.make_async_copy(msg.at[b], agg.at[cidx.at[b]], ssems[b]).wait()

    pidx(0, 0)
    pidx(1, 1)

    def zb(i, _):
      zbuf[i, :] = jnp.zeros((F,), jnp.float32)
      return 0
    lax.fori_loop(0, ZROWS, zb, 0)

    if not with_gather:
      def ob(i, _):
        for b in range(2):
          msg[b, i, :] = jnp.ones((F,), jnp.float32)
        return 0
      lax.fori_loop(0, GSZ, ob, 0)

    def zs(i, _):
      pltpu.sync_copy(zbuf, agg.at[pl.ds(s * RPT + i * ZROWS, ZROWS)])
      return 0
    lax.fori_loop(0, RPT // ZROWS, zs, 0)
    plsc.subcore_barrier()

    widx(0)
    gat(0)
    wgat(0)
    sca(0)
    widx(1)
    gat(1)
    wsca(0)
    pidx(0, 2)
    wgat(1)
    sca(1)

    def body(i, _):
      g = 2 * i + 2
      widx(0)
      gat(0)
      wsca(1)
      pidx(1, g + 1)
      wgat(0)
      sca(0)
      widx(1)
      gat(1)
      wsca(0)
      pidx(0, g + 2)
      wgat(1)
      sca(1)
      return 0
    lax.fori_loop(0, (gw - 2) // 2, body, 0)

    wsca(1)
    widx(0)

    plsc.subcore_barrier()
    pltpu.sync_copy(agg.at[pl.ds(s * RPT, RPT)],
                    out_hbm.at[c, pl.ds(s * RPT, RPT)])

  return k(h, row2d, col2d)


def _tc_layer1(xp2, degp_r, W1i, W1r, b1t):
  def body(x_ref, dg_ref, w1i_ref, w1r_ref, b1_ref, h1p_ref, root1_ref,
           dinv_ref):
    deg = dg_ref[0] + dg_ref[1]
    dinv = jnp.where(deg > 0, lax.rsqrt(jnp.maximum(deg, 1.0)), 0.0)
    xb = x_ref[...]
    h1p_ref[...] = jnp.dot(xb, w1i_ref[...],
                           preferred_element_type=jnp.float32) * dinv
    root1_ref[...] = jnp.dot(xb, w1r_ref[...],
                             preferred_element_type=jnp.float32) + b1_ref[...]
    dinv_ref[...] = dinv

  return pl.pallas_call(
      body,
      grid=(PALL // BP,),
      in_specs=[
          pl.BlockSpec((BP, 128), lambda i: (i, 0)),
          pl.BlockSpec((NCORE, BP, 128), lambda i: (0, i, 0)),
          pl.BlockSpec((128, 128), lambda i: (0, 0)),
          pl.BlockSpec((128, 128), lambda i: (0, 0)),
          pl.BlockSpec((1, 128), lambda i: (0, 0)),
      ],
      out_specs=[pl.BlockSpec((BP, 128), lambda i: (i, 0))] * 3,
      out_shape=[jax.ShapeDtypeStruct((PALL, 128), jnp.float32)] * 3,
  )(xp2, degp_r, W1i, W1r, b1t)


def _tc_layer2(aggp_r, dinv, root1, W2i, W2r, b2t):
  def body(ag_ref, dinv_ref, root1_ref, w2i_ref, w2r_ref, b2_ref,
           h2p_ref, root2_ref):
    dv = dinv_ref[...]
    out1 = jax.nn.relu((ag_ref[0] + ag_ref[1]) * dv + root1_ref[...])
    h2p_ref[...] = jnp.dot(out1, w2i_ref[...],
                           preferred_element_type=jnp.float32) * dv
    root2_ref[...] = jnp.dot(out1, w2r_ref[...],
                             preferred_element_type=jnp.float32) + b2_ref[...]

  return pl.pallas_call(
      body,
      grid=(PALL // BP,),
      in_specs=[
          pl.BlockSpec((NCORE, BP, 128), lambda i: (0, i, 0)),
          pl.BlockSpec((BP, 128), lambda i: (i, 0)),
          pl.BlockSpec((BP, 128), lambda i: (i, 0)),
          pl.BlockSpec((128, 128), lambda i: (0, 0)),
          pl.BlockSpec((128, 128), lambda i: (0, 0)),
          pl.BlockSpec((1, 128), lambda i: (0, 0)),
      ],
      out_specs=[pl.BlockSpec((BP, 128), lambda i: (i, 0))] * 2,
      out_shape=[jax.ShapeDtypeStruct((PALL, 128), jnp.float32)] * 2,
  )(aggp_r, dinv, root1, W2i, W2r, b2t)


def _tc_final(aggp_r, dinv, root2):
  def body(ag_ref, dinv_ref, root2_ref, out_ref):
    out_ref[...] = jax.nn.relu(
        (ag_ref[0] + ag_ref[1]) * dinv_ref[...] + root2_ref[...])

  return pl.pallas_call(
      body,
      grid=(PALL // BP,),
      in_specs=[
          pl.BlockSpec((NCORE, BP, 128), lambda i: (0, i, 0)),
          pl.BlockSpec((BP, 128), lambda i: (i, 0)),
          pl.BlockSpec((BP, 128), lambda i: (i, 0)),
      ],
      out_specs=pl.BlockSpec((BP, 128), lambda i: (i, 0)),
      out_shape=jax.ShapeDtypeStruct((PALL, 128), jnp.float32),
  )(aggp_r, dinv, root2)


def kernel(x, edge_index, w1_init, w1_root, b1, w2_init, w2_root, b2):
  rowp = edge_index[0].reshape(EG, GSZ)
  colp = edge_index[1].reshape(EG, GSZ)

  eye8 = jnp.eye(8, dtype=jnp.float32)
  w1ip = jnp.zeros((16, F), jnp.float32).at[:8].set(w1_init)
  w1rp = jnp.zeros((16, F), jnp.float32).at[:8].set(w1_root)
  W1i = jnp.kron(eye8, w1ip)
  W1r = jnp.kron(eye8, w1rp)
  W2i = jnp.kron(eye8, w2_init)
  W2r = jnp.kron(eye8, w2_root)
  b1t = jnp.tile(b1, 8).reshape(1, 128)
  b2t = jnp.tile(b2, 8).reshape(1, 128)
  xp2 = jnp.pad(x.reshape(P, 8, 8), ((0, PALL - P), (0, 0), (0, 8)),
                ).reshape(PALL, 128)

  dummy_h = jnp.zeros((1, F), jnp.float32)
  degp = _edge_pass(dummy_h, rowp, colp, with_gather=False)
  h1p, root1, dinv = _tc_layer1(xp2, degp.reshape(NCORE, PALL, 128),
                                W1i, W1r, b1t)
  aggp1 = _edge_pass(h1p.reshape(NPAD, F), rowp, colp, with_gather=True)
  h2p, root2 = _tc_layer2(aggp1.reshape(NCORE, PALL, 128), dinv, root1,
                          W2i, W2r, b2t)
  aggp2 = _edge_pass(h2p.reshape(NPAD, F), rowp, colp, with_gather=True)
  out = _tc_final(aggp2.reshape(NCORE, PALL, 128), dinv, root2)
  return out[:P].reshape(N, F)


# speedup vs baseline: 1.0861x; 1.0861x over previous
import functools

import jax
import jax.numpy as jnp
from jax import lax
from jax.experimental import pallas as pl
from jax.experimental.pallas import tpu as pltpu
from jax.experimental.pallas import tpu_sc as plsc

N = 100000
E = 3200000
F = 16

CHUNK = 128
K = 5
GSZ = K * CHUNK
NCORE, NSUB = 2, 16
NW = NCORE * NSUB
EG = E // GSZ
NBIG = 4
GBIG, GSMALL = 158, 156
NPAD = 100096
RPT = NPAD // NSUB
ZROWS = 391

P = N * F // 128
PALL = NPAD * F // 128
BP = 3128


def _edge_pass(h, row2d, col2d, with_gather):
  mesh = plsc.VectorSubcoreMesh(core_axis_name="c", subcore_axis_name="s")

  @functools.partial(
      pl.kernel,
      out_type=jax.ShapeDtypeStruct((NCORE, NPAD, F), jnp.float32),
      mesh=mesh,
      scratch_types=[
          pltpu.VMEM((2, GSZ), jnp.int32),
          pltpu.VMEM((2, GSZ), jnp.int32),
          pltpu.VMEM((2, GSZ, F), jnp.float32),
          pltpu.VMEM((ZROWS, F), jnp.float32),
          pltpu.VMEM_SHARED((NPAD, F), jnp.float32),
          pltpu.SemaphoreType.DMA,
          pltpu.SemaphoreType.DMA,
          pltpu.SemaphoreType.DMA,
          pltpu.SemaphoreType.DMA,
          pltpu.SemaphoreType.DMA,
          pltpu.SemaphoreType.DMA,
      ],
      compiler_params=pltpu.CompilerParams(use_tc_tiling_on_sc=False),
  )
  def k(h_hbm, row_hbm, col_hbm, out_hbm, ridx, cidx, msg, zbuf, agg,
        gsem0, gsem1, ssem0, ssem1, isem0, isem1):
    c = lax.axis_index("c")
    s = lax.axis_index("s")
    wid = s * NCORE + c

    gw = GBIG - 2 * (wid >= NBIG).astype(jnp.int32)
    base = GSMALL * wid + 2 * jnp.minimum(wid, NBIG)
    gmax = base + gw - 1
    gsems = (gsem0, gsem1)
    ssems = (ssem0, ssem1)
    isems = (isem0, isem1)

    def pidx(b, g):
      off = jnp.minimum(base + g, gmax)
      if with_gather:
        pltpu.make_async_copy(row_hbm.at[off], ridx.at[b], isems[b]).start()
      pltpu.make_async_copy(col_hbm.at[off], cidx.at[b], isems[b]).start()

    def widx(b):
      if with_gather:
        pltpu.make_async_copy(row_hbm.at[0], ridx.at[b], isems[b]).wait()
      pltpu.make_async_copy(col_hbm.at[0], cidx.at[b], isems[b]).wait()

    def gat(b):
      if with_gather:
        pltpu.make_async_copy(h_hbm.at[ridx.at[b]], msg.at[b],
                              gsems[b]).start()

    def wgat(b):
      if with_gather:
        pltpu.make_async_copy(h_hbm.at[ridx.at[b]], msg.at[b],
                              gsems[b]).wait()

    def sca(b):
      pltpu.make_async_copy(msg.at[b], agg.at[cidx.at[b]],
                            ssems[b]).start(add=True)

    def wsca(b):
      pltpu.make_async_copy(msg.at[b], agg.at[cidx.at[b]], ssems[b]).wait()

    pidx(0, 0)
    pidx(1, 1)

    def zb(i, _):
      zbuf[i, :] = jnp.zeros((F,), jnp.float32)
      return 0
    lax.fori_loop(0, ZROWS, zb, 0)

    if not with_gather:
      def ob(i, _):
        for b in range(2):
          msg[b, i, :] = jnp.ones((F,), jnp.float32)
        return 0
      lax.fori_loop(0, GSZ, ob, 0)

    def zs(i, _):
      pltpu.sync_copy(zbuf, agg.at[pl.ds(s * RPT + i * ZROWS, ZROWS)])
      return 0
    lax.fori_loop(0, RPT // ZROWS, zs, 0)
    plsc.subcore_barrier()

    widx(0)
    gat(0)
    wgat(0)
    sca(0)
    widx(1)
    gat(1)
    wsca(0)
    pidx(0, 2)
    wgat(1)
    sca(1)

    def body(i, _):
      g = 2 * i + 2
      widx(0)
      gat(0)
      wsca(1)
      pidx(1, g + 1)
      wgat(0)
      sca(0)
      widx(1)
      gat(1)
      wsca(0)
      pidx(0, g + 2)
      wgat(1)
      sca(1)
      return 0
    lax.fori_loop(0, (gw - 2) // 2, body, 0)

    wsca(1)
    widx(0)

    plsc.subcore_barrier()
    pltpu.sync_copy(agg.at[pl.ds(s * RPT, RPT)],
                    out_hbm.at[c, pl.ds(s * RPT, RPT)])

  return k(h, row2d, col2d)


def _tc_layer1(xp2, degp_r, W1i, W1r, b1t):
  def body(x_ref, dg_ref, w1i_ref, w1r_ref, b1_ref, h1p_ref, root1_ref,
           dinv_ref):
    deg = dg_ref[0] + dg_ref[1]
    dinv = jnp.where(deg > 0, lax.rsqrt(jnp.maximum(deg, 1.0)), 0.0)
    xb = x_ref[...]
    h1p_ref[...] = jnp.dot(xb, w1i_ref[...],
                           preferred_element_type=jnp.float32) * dinv
    root1_ref[...] = jnp.dot(xb, w1r_ref[...],
                             preferred_element_type=jnp.float32) + b1_ref[...]
    dinv_ref[...] = dinv

  return pl.pallas_call(
      body,
      grid=(PALL // BP,),
      in_specs=[
          pl.BlockSpec((BP, 128), lambda i: (i, 0)),
          pl.BlockSpec((NCORE, BP, 128), lambda i: (0, i, 0)),
          pl.BlockSpec((128, 128), lambda i: (0, 0)),
          pl.BlockSpec((128, 128), lambda i: (0, 0)),
          pl.BlockSpec((1, 128), lambda i: (0, 0)),
      ],
      out_specs=[pl.BlockSpec((BP, 128), lambda i: (i, 0))] * 3,
      out_shape=[jax.ShapeDtypeStruct((PALL, 128), jnp.float32)] * 3,
  )(xp2, degp_r, W1i, W1r, b1t)


def _tc_layer2(aggp_r, dinv, root1, W2i, W2r, b2t):
  def body(ag_ref, dinv_ref, root1_ref, w2i_ref, w2r_ref, b2_ref,
           h2p_ref, root2_ref):
    dv = dinv_ref[...]
    out1 = jax.nn.relu((ag_ref[0] + ag_ref[1]) * dv + root1_ref[...])
    h2p_ref[...] = jnp.dot(out1, w2i_ref[...],
                           preferred_element_type=jnp.float32) * dv
    root2_ref[...] = jnp.dot(out1, w2r_ref[...],
                             preferred_element_type=jnp.float32) + b2_ref[...]

  return pl.pallas_call(
      body,
      grid=(PALL // BP,),
      in_specs=[
          pl.BlockSpec((NCORE, BP, 128), lambda i: (0, i, 0)),
          pl.BlockSpec((BP, 128), lambda i: (i, 0)),
          pl.BlockSpec((BP, 128), lambda i: (i, 0)),
          pl.BlockSpec((128, 128), lambda i: (0, 0)),
          pl.BlockSpec((128, 128), lambda i: (0, 0)),
          pl.BlockSpec((1, 128), lambda i: (0, 0)),
      ],
      out_specs=[pl.BlockSpec((BP, 128), lambda i: (i, 0))] * 2,
      out_shape=[jax.ShapeDtypeStruct((PALL, 128), jnp.float32)] * 2,
  )(aggp_r, dinv, root1, W2i, W2r, b2t)


def _tc_final(aggp_r, dinv, root2):
  def body(ag_ref, dinv_ref, root2_ref, out_ref):
    out_ref[...] = jax.nn.relu(
        (ag_ref[0] + ag_ref[1]) * dinv_ref[...] + root2_ref[...])

  return pl.pallas_call(
      body,
      grid=(PALL // BP,),
      in_specs=[
          pl.BlockSpec((NCORE, BP, 128), lambda i: (0, i, 0)),
          pl.BlockSpec((BP, 128), lambda i: (i, 0)),
          pl.BlockSpec((BP, 128), lambda i: (i, 0)),
      ],
      out_specs=pl.BlockSpec((BP, 128), lambda i: (i, 0)),
      out_shape=jax.ShapeDtypeStruct((PALL, 128), jnp.float32),
  )(aggp_r, dinv, root2)


def kernel(x, edge_index, w1_init, w1_root, b1, w2_init, w2_root, b2):
  rowp = edge_index[0].reshape(EG, GSZ)
  colp = edge_index[1].reshape(EG, GSZ)

  eye8 = jnp.eye(8, dtype=jnp.float32)
  w1ip = jnp.zeros((16, F), jnp.float32).at[:8].set(w1_init)
  w1rp = jnp.zeros((16, F), jnp.float32).at[:8].set(w1_root)
  W1i = jnp.kron(eye8, w1ip)
  W1r = jnp.kron(eye8, w1rp)
  W2i = jnp.kron(eye8, w2_init)
  W2r = jnp.kron(eye8, w2_root)
  b1t = jnp.tile(b1, 8).reshape(1, 128)
  b2t = jnp.tile(b2, 8).reshape(1, 128)
  xp2 = jnp.pad(x.reshape(P, 8, 8), ((0, PALL - P), (0, 0), (0, 8)),
                ).reshape(PALL, 128)

  dummy_h = jnp.zeros((1, F), jnp.float32)
  degp = _edge_pass(dummy_h, rowp, colp, with_gather=False)
  h1p, root1, dinv = _tc_layer1(xp2, degp.reshape(NCORE, PALL, 128),
                                W1i, W1r, b1t)
  aggp1 = _edge_pass(h1p.reshape(NPAD, F), rowp, colp, with_gather=True)
  h2p, root2 = _tc_layer2(aggp1.reshape(NCORE, PALL, 128), dinv, root1,
                          W2i, W2r, b2t)
  aggp2 = _edge_pass(h2p.reshape(NPAD, F), rowp, colp, with_gather=True)
  out = _tc_final(aggp2.reshape(NCORE, PALL, 128), dinv, root2)
  return out[:P].reshape(N, F)
